# baseline (speedup 1.0000x reference)
import functools

import jax
import jax.numpy as jnp
from jax import lax
from jax.experimental import pallas as pl
from jax.experimental.pallas import tpu as pltpu
from jax.experimental.pallas import tpu_sc as plsc

B = 4096
L = 200
EMB = 64
TOTAL = B * L
NUM_CORES = 2
NUM_SUBCORES = 16
NW = NUM_CORES * NUM_SUBCORES
PER_W = TOTAL // NW
CHUNK = 128
NCHUNK = PER_W // CHUNK

_mesh = plsc.VectorSubcoreMesh(core_axis_name="c", subcore_axis_name="s")


@functools.partial(
    pl.kernel,
    out_type=jax.ShapeDtypeStruct((TOTAL, EMB), jnp.float32),
    mesh=_mesh,
    scratch_types=[
        pltpu.VMEM((NCHUNK, CHUNK), jnp.int32),
        pltpu.VMEM((CHUNK, EMB), jnp.float32),
        pltpu.SemaphoreType.DMA,
    ],
    compiler_params=pltpu.CompilerParams(use_tc_tiling_on_sc=False),
)
def _emb_lookup(idx_hbm, table_hbm, out_hbm, idx_v, rows_v, sem):
    wid = lax.axis_index("s") * NUM_CORES + lax.axis_index("c")
    pltpu.sync_copy(idx_hbm.at[wid], idx_v)
    base = wid * PER_W

    def body(j, carry):
        pltpu.async_copy(table_hbm.at[idx_v.at[j]], rows_v, sem).wait()
        pltpu.sync_copy(rows_v, out_hbm.at[pl.ds(base + j * CHUNK, CHUNK)])
        return carry

    lax.fori_loop(0, NCHUNK, body, 0)


def kernel(x, table):
    idx = x.reshape(NW, NCHUNK, CHUNK).astype(jnp.int32)
    out = _emb_lookup(idx, table)
    return out.reshape(B, L, EMB)


# speedup vs baseline: 1.1110x; 1.1110x over previous
import functools

import jax
import jax.numpy as jnp
from jax import lax
from jax.experimental import pallas as pl
from jax.experimental.pallas import tpu as pltpu
from jax.experimental.pallas import tpu_sc as plsc

B = 4096
L = 200
EMB = 64
TOTAL = B * L
NUM_CORES = 2
NUM_SUBCORES = 16
NW = NUM_CORES * NUM_SUBCORES
PER_W = TOTAL // NW
CHUNK = 128
NCHUNK = PER_W // CHUNK
K = 2
ROWS_PER_ROUND = K * CHUNK
R = NCHUNK // K
NB = 4

_mesh = plsc.VectorSubcoreMesh(core_axis_name="c", subcore_axis_name="s")


@functools.partial(
    pl.kernel,
    out_type=jax.ShapeDtypeStruct((TOTAL, EMB), jnp.float32),
    mesh=_mesh,
    scratch_types=(
        [pltpu.VMEM((NCHUNK, CHUNK), jnp.int32)]
        + [pltpu.VMEM((ROWS_PER_ROUND, EMB), jnp.float32)
           for _ in range(NB)]
        + [pltpu.SemaphoreType.DMA for _ in range(2 * NB)]
    ),
    compiler_params=pltpu.CompilerParams(use_tc_tiling_on_sc=False),
)
def _emb_lookup(idx_hbm, table_hbm, out_hbm, idx_v,
                bank0, bank1, bank2, bank3,
                g0, g1, g2, g3, o0, o1, o2, o3):
    banks = (bank0, bank1, bank2, bank3)
    gsems = (g0, g1, g2, g3)
    osems = (o0, o1, o2, o3)

    wid = lax.axis_index("s") * NUM_CORES + lax.axis_index("c")
    pltpu.sync_copy(idx_hbm.at[wid], idx_v)
    base = wid * PER_W

    def fire_gathers(r, bi):
        for b in range(K):
            pltpu.async_copy(table_hbm.at[idx_v.at[r * K + b]],
                             banks[bi].at[pl.ds(b * CHUNK, CHUNK)], gsems[bi])

    def drain_gathers(bi):
        pltpu.make_async_copy(table_hbm.at[pl.ds(0, ROWS_PER_ROUND)],
                              banks[bi], gsems[bi]).wait()

    def fire_write(r, bi):
        pltpu.async_copy(banks[bi],
                         out_hbm.at[pl.ds(base + r * ROWS_PER_ROUND,
                                          ROWS_PER_ROUND)], osems[bi])

    def drain_write(bi):
        pltpu.make_async_copy(banks[bi],
                              out_hbm.at[pl.ds(0, ROWS_PER_ROUND)],
                              osems[bi]).wait()

    def do_round(r, bi, drain_w=True, fire_g=True):
        ob = (bi + 2) % NB
        if drain_w:
            drain_write(ob)
        if fire_g:
            fire_gathers(r + 2, ob)
        drain_gathers(bi)
        fire_write(r, bi)

    fire_gathers(0, 0)
    fire_gathers(1, 1)
    do_round(0, 0, drain_w=False)
    do_round(1, 1, drain_w=False)

    @pl.loop(2, R - 2, step=NB)
    def _rounds(r0):
        do_round(r0, 2)
        do_round(r0 + 1, 3)
        do_round(r0 + 2, 0)
        do_round(r0 + 3, 1)

    do_round(R - 2, 2, fire_g=False)
    do_round(R - 1, 3, fire_g=False)
    drain_write(2)
    drain_write(3)


def kernel(x, table):
    idx = x.reshape(NW, NCHUNK, CHUNK).astype(jnp.int32)
    out = _emb_lookup(idx, table)
    return out.reshape(B, L, EMB)


# speedup vs baseline: 1.1397x; 1.0258x over previous
import functools

import jax
import jax.numpy as jnp
from jax import lax
from jax.experimental import pallas as pl
from jax.experimental.pallas import tpu as pltpu
from jax.experimental.pallas import tpu_sc as plsc

B = 4096
L = 200
EMB = 64
TOTAL = B * L
NUM_CORES = 2
NUM_SUBCORES = 16
NW = NUM_CORES * NUM_SUBCORES
PER_W = TOTAL // NW
CHUNK = 128
NCHUNK = PER_W // CHUNK
K = 2
ROWS_PER_ROUND = K * CHUNK
R = NCHUNK // K
NB = 4

_mesh = plsc.VectorSubcoreMesh(core_axis_name="c", subcore_axis_name="s")


@functools.partial(
    pl.kernel,
    out_type=jax.ShapeDtypeStruct((TOTAL, EMB), jnp.float32),
    mesh=_mesh,
    scratch_types=(
        [pltpu.VMEM((NCHUNK, CHUNK), jnp.int32)]
        + [pltpu.VMEM((ROWS_PER_ROUND, EMB), jnp.float32)
           for _ in range(NB)]
        + [pltpu.SemaphoreType.DMA for _ in range(2 * NB)]
    ),
    compiler_params=pltpu.CompilerParams(use_tc_tiling_on_sc=False),
)
def _emb_lookup(idx_hbm, table_hbm, out_hbm, idx_v,
                bank0, bank1, bank2, bank3,
                g0, g1, g2, g3, o0, o1, o2, o3):
    banks = (bank0, bank1, bank2, bank3)
    gsems = (g0, g1, g2, g3)
    osems = (o0, o1, o2, o3)

    wid = lax.axis_index("s") * NUM_CORES + lax.axis_index("c")
    pltpu.sync_copy(idx_hbm.at[wid], idx_v)
    base = wid * PER_W

    def fire_gathers(r, bi):
        for b in range(K):
            pltpu.async_copy(table_hbm.at[idx_v.at[r * K + b]],
                             banks[bi].at[pl.ds(b * CHUNK, CHUNK)], gsems[bi])

    def drain_gathers(bi):
        pltpu.make_async_copy(table_hbm.at[pl.ds(0, ROWS_PER_ROUND)],
                              banks[bi], gsems[bi]).wait()

    def fire_write(r, bi):
        pltpu.async_copy(banks[bi],
                         out_hbm.at[pl.ds(base + r * ROWS_PER_ROUND,
                                          ROWS_PER_ROUND)], osems[bi])

    def drain_write(bi):
        pltpu.make_async_copy(banks[bi],
                              out_hbm.at[pl.ds(0, ROWS_PER_ROUND)],
                              osems[bi]).wait()

    def do_round(r, bi, drain_w=True, fire_g=True):
        ob = (bi + 2) % NB
        if drain_w:
            drain_write(ob)
        if fire_g:
            fire_gathers(r + 2, ob)
        drain_gathers(bi)
        fire_write(r, bi)

    fire_gathers(0, 0)
    fire_gathers(1, 1)
    do_round(0, 0, drain_w=False)
    do_round(1, 1, drain_w=False)

    @pl.loop(2, R - 2, step=NB)
    def _rounds(r0):
        do_round(r0, 2)
        do_round(r0 + 1, 3)
        do_round(r0 + 2, 0)
        do_round(r0 + 3, 1)

    do_round(R - 2, 2, fire_g=False)
    do_round(R - 1, 3, fire_g=False)
    drain_write(2)
    drain_write(3)


def kernel(x, table):
    idx = x.T.reshape(NW, NCHUNK, CHUNK).astype(jnp.int32)
    out = _emb_lookup(idx, table)
    return jnp.transpose(out.reshape(L, B, EMB), (1, 0, 2))
